# baseline (speedup 1.0000x reference)
import functools

import jax
import jax.numpy as jnp
from jax import lax
from jax.experimental import pallas as pl
from jax.experimental.pallas import tpu as pltpu
from jax.experimental.pallas import tpu_sc as plsc

NC = 2
NS = 16
LANES = 16
NW = NC * NS


def _agg_body(x_hbm, src_hbm, dst_hbm, out_hbm,
              sidx, didx0, didx1, rows0, rows1, acc, sem0, sem1,
              *, bsz, nb, stride, wlen):
    c = lax.axis_index("c")
    s = lax.axis_index("s")
    d = x_hbm.shape[1]
    rows = (rows0, rows1)
    didx = (didx0, didx1)
    sems = (sem0, sem1)
    wid = c * NS + s

    zero = jnp.zeros((LANES,), jnp.float32)
    for r in range(bsz):
        for cc in range(d // LANES):
            rows0[r, pl.ds(cc * LANES, LANES)] = zero
    base = s * stride
    nz = -(-wlen // bsz)
    for k in range(nz):
        off = min(k * bsz, wlen - bsz)
        pltpu.sync_copy(rows0, acc.at[pl.ds(base + off, bsz)])
    plsc.subcore_barrier()

    epw = nb * bsz
    ebase = wid * epw
    pltpu.sync_copy(src_hbm.at[pl.ds(ebase, epw)], sidx)

    def fire(j, b):
        pltpu.make_async_copy(
            x_hbm.at[sidx.at[pl.ds(j * bsz, bsz)]], rows[b], sems[b]).start()
        pltpu.make_async_copy(
            dst_hbm.at[pl.ds(ebase + j * bsz, bsz)], didx[b], sems[b]).start()

    def drain_scatter(j, b):
        pltpu.make_async_copy(
            x_hbm.at[sidx.at[pl.ds(j * bsz, bsz)]], rows[b], sems[b]).wait()
        pltpu.make_async_copy(
            dst_hbm.at[pl.ds(ebase + j * bsz, bsz)], didx[b], sems[b]).wait()
        pltpu.sync_copy(rows[b], acc.at[didx[b]], add=True)

    for b in range(2):
        fire(b, b)
    main = (nb - 2) // 2 * 2

    @pl.loop(0, main, step=2)
    def _(g):
        for b in range(2):
            j = g + b
            drain_scatter(j, b)
            fire(j + 2, b)

    for j in range(main, nb):
        drain_scatter(j, j % 2)
        if j + 2 < nb:
            fire(j + 2, j % 2)
    plsc.subcore_barrier()

    pltpu.sync_copy(acc.at[pl.ds(base, wlen)], out_hbm.at[c, pl.ds(base, wlen)])


def _aggregate(x, src, dst, nb, bsz):
    n, d = x.shape
    stride = (n // NS) // 8 * 8
    wlen = n - (NS - 1) * stride
    n_acc = n + 8
    body = functools.partial(_agg_body, bsz=bsz, nb=nb,
                             stride=stride, wlen=wlen)
    return pl.kernel(
        body,
        out_type=jax.ShapeDtypeStruct((NC, n, d), jnp.float32),
        mesh=plsc.VectorSubcoreMesh(core_axis_name="c", subcore_axis_name="s"),
        scratch_types=[
            pltpu.VMEM((nb * bsz,), jnp.int32),
            pltpu.VMEM((bsz,), jnp.int32),
            pltpu.VMEM((bsz,), jnp.int32),
            pltpu.VMEM((bsz, d), jnp.float32),
            pltpu.VMEM((bsz, d), jnp.float32),
            pltpu.VMEM_SHARED((n_acc, d), jnp.float32),
            pltpu.SemaphoreType.DMA,
            pltpu.SemaphoreType.DMA,
        ],
    )(x, src, dst)


def _mm_body(x_ref, p_ref, w_ref, b_ref, z_ref, s_ref):
    i = pl.program_id(0)
    g = pl.num_programs(0)
    rst = x_ref[...] + p_ref[0] + p_ref[1]
    z = lax.dot_general(rst, w_ref[...], (((1,), (1,)), ((), ())),
                        preferred_element_type=jnp.float32)
    z = z + b_ref[...][None, :]
    z_ref[...] = z

    @pl.when(i == 0)
    def _():
        s_ref[...] = jnp.zeros_like(s_ref)

    colsum = jnp.sum(z, axis=0, keepdims=True)
    colsq = jnp.sum(z * z, axis=0, keepdims=True)
    s_ref[...] += jnp.concatenate([colsum, colsq], axis=0)


def _layer_mm(x, p, W, b):
    n, d = x.shape
    hdim = W.shape[0]
    r = 2000
    grid = (n // r,)
    return pl.pallas_call(
        _mm_body,
        grid=grid,
        in_specs=[
            pl.BlockSpec((r, d), lambda i: (i, 0)),
            pl.BlockSpec((NC, r, d), lambda i: (0, i, 0)),
            pl.BlockSpec((hdim, d), lambda i: (0, 0)),
            pl.BlockSpec((hdim,), lambda i: (0,)),
        ],
        out_specs=[
            pl.BlockSpec((r, hdim), lambda i: (i, 0)),
            pl.BlockSpec((2, hdim), lambda i: (0, 0)),
        ],
        out_shape=[
            jax.ShapeDtypeStruct((n, hdim), jnp.float32),
            jax.ShapeDtypeStruct((2, hdim), jnp.float32),
        ],
    )(x, p, W, b)


def _bn_body(z_ref, s_ref, g_ref, be_ref, o_ref, *, n):
    mu = s_ref[0:1, :] * (1.0 / n)
    ex2 = s_ref[1:2, :] * (1.0 / n)
    var = ex2 - mu * mu
    inv = lax.rsqrt(var + 1e-5)
    zn = (z_ref[...] - mu) * (inv * g_ref[...][None, :]) + be_ref[...][None, :]
    o_ref[...] = jnp.maximum(zn, 0.0)


def _bn_relu(z, stats, g, be):
    n, hdim = z.shape
    r = 2000
    return pl.pallas_call(
        functools.partial(_bn_body, n=n),
        grid=(n // r,),
        in_specs=[
            pl.BlockSpec((r, hdim), lambda i: (i, 0)),
            pl.BlockSpec((2, hdim), lambda i: (0, 0)),
            pl.BlockSpec((hdim,), lambda i: (0,)),
            pl.BlockSpec((hdim,), lambda i: (0,)),
        ],
        out_specs=pl.BlockSpec((r, hdim), lambda i: (i, 0)),
        out_shape=jax.ShapeDtypeStruct((n, hdim), jnp.float32),
    )(z, stats, g, be)


def _head_body(h_ref, w1_ref, b1_ref, w2_ref, b2_ref, o_ref, acc_ref, *, n):
    i = pl.program_id(0)
    g = pl.num_programs(0)

    @pl.when(i == 0)
    def _():
        acc_ref[...] = jnp.zeros_like(acc_ref)

    acc_ref[...] += jnp.sum(h_ref[...], axis=0, keepdims=True)

    @pl.when(i == g - 1)
    def _():
        hg = acc_ref[...] * (1.0 / n)
        y = lax.dot_general(hg, w1_ref[...], (((1,), (1,)), ((), ())),
                            preferred_element_type=jnp.float32)
        y = y + b1_ref[...][None, :]
        y = jnp.where(y > 0, y, jnp.exp(y) - 1.0)
        y2 = lax.dot_general(y, w2_ref[...], (((1,), (1,)), ((), ())),
                             preferred_element_type=jnp.float32)
        y2 = y2 + b2_ref[...][None, :]
        m = jnp.max(y2, axis=0, keepdims=True)
        lse = m + jnp.log(jnp.sum(jnp.exp(y2 - m), axis=0, keepdims=True))
        o_ref[...] = y2 - lse


def _head(hfin, fc1W, fc1b, fc2W, fc2b):
    n, hdim = hfin.shape
    cdim = fc2W.shape[0]
    r = 2000
    return pl.pallas_call(
        functools.partial(_head_body, n=n),
        grid=(n // r,),
        in_specs=[
            pl.BlockSpec((r, hdim), lambda i: (i, 0)),
            pl.BlockSpec((hdim, hdim), lambda i: (0, 0)),
            pl.BlockSpec((hdim,), lambda i: (0,)),
            pl.BlockSpec((cdim, hdim), lambda i: (0, 0)),
            pl.BlockSpec((cdim,), lambda i: (0,)),
        ],
        out_specs=pl.BlockSpec((1, cdim), lambda i: (0, 0)),
        out_shape=jax.ShapeDtypeStruct((1, cdim), jnp.float32),
        scratch_shapes=[pltpu.VMEM((1, hdim), jnp.float32)],
    )(hfin, fc1W, fc1b, fc2W, fc2b)


def kernel(h, edge_index, W0, b0, g0, be0, W1, b1, g1, be1, W2, b2, g2, be2,
           W3, b3, g3, be3, fc1W, fc1b, fc2W, fc2b):
    n = h.shape[0]
    e = edge_index.shape[1]
    bsz = 104
    nb = -(-e // (NW * bsz))
    epw = nb * bsz
    pad = NW * epw - e
    src = jnp.concatenate([edge_index[0], jnp.zeros((pad,), edge_index.dtype)])
    dst = jnp.concatenate([edge_index[1], jnp.full((pad,), n, edge_index.dtype)])
    x = h
    for W, b, g, be in ((W0, b0, g0, be0), (W1, b1, g1, be1),
                        (W2, b2, g2, be2), (W3, b3, g3, be3)):
        p = _aggregate(x, src, dst, nb, bsz)
        z, stats = _layer_mm(x, p, W, b)
        x = _bn_relu(z, stats, g, be)
    return _head(x, fc1W, fc1b, fc2W, fc2b)


# speedup vs baseline: 1.7003x; 1.7003x over previous
import functools

import jax
import jax.numpy as jnp
from jax import lax
from jax.experimental import pallas as pl
from jax.experimental.pallas import tpu as pltpu
from jax.experimental.pallas import tpu_sc as plsc

NC = 2
NS = 16
LANES = 16
NW = NC * NS


def _agg_body(x_hbm, src_hbm, dst_hbm, out_hbm,
              sidx, didx0, didx1, rows0, rows1, acc, sem0, sem1,
              *, bsz, nb, stride, wlen):
    c = lax.axis_index("c")
    s = lax.axis_index("s")
    d = x_hbm.shape[1]
    rows = (rows0, rows1)
    didx = (didx0, didx1)
    sems = (sem0, sem1)
    wid = c * NS + s

    zero = jnp.zeros((LANES,), jnp.float32)
    for r in range(bsz):
        for cc in range(d // LANES):
            rows0[r, pl.ds(cc * LANES, LANES)] = zero
    base = s * stride
    nz = -(-wlen // bsz)
    for k in range(nz):
        off = min(k * bsz, wlen - bsz)
        pltpu.sync_copy(rows0, acc.at[pl.ds(base + off, bsz)])
    plsc.subcore_barrier()

    epw = nb * bsz
    ebase = wid * epw
    pltpu.sync_copy(src_hbm.at[pl.ds(ebase, epw)], sidx)

    def fire(j, b):
        pltpu.make_async_copy(
            x_hbm.at[sidx.at[pl.ds(j * bsz, bsz)]], rows[b], sems[b]).start()
        pltpu.make_async_copy(
            dst_hbm.at[pl.ds(ebase + j * bsz, bsz)], didx[b], sems[b]).start()

    def drain_scatter(j, b):
        pltpu.make_async_copy(
            x_hbm.at[sidx.at[pl.ds(j * bsz, bsz)]], rows[b], sems[b]).wait()
        pltpu.make_async_copy(
            dst_hbm.at[pl.ds(ebase + j * bsz, bsz)], didx[b], sems[b]).wait()
        pltpu.sync_copy(rows[b], acc.at[didx[b]], add=True)

    for b in range(2):
        fire(b, b)
    main = (nb - 2) // 2 * 2

    @pl.loop(0, main, step=2)
    def _(g):
        for b in range(2):
            j = g + b
            drain_scatter(j, b)
            fire(j + 2, b)

    for j in range(main, nb):
        drain_scatter(j, j % 2)
        if j + 2 < nb:
            fire(j + 2, j % 2)
    plsc.subcore_barrier()

    pltpu.sync_copy(acc.at[pl.ds(base, wlen)], out_hbm.at[c, pl.ds(base, wlen)])


def _aggregate(x, src, dst, nb, bsz):
    n, d = x.shape
    stride = (n // NS) // 8 * 8
    wlen = n - (NS - 1) * stride
    n_acc = n + 8
    body = functools.partial(_agg_body, bsz=bsz, nb=nb,
                             stride=stride, wlen=wlen)
    return pl.kernel(
        body,
        out_type=jax.ShapeDtypeStruct((NC, n, d), jnp.float32),
        mesh=plsc.VectorSubcoreMesh(core_axis_name="c", subcore_axis_name="s"),
        scratch_types=[
            pltpu.VMEM((nb * bsz,), jnp.int32),
            pltpu.VMEM((bsz,), jnp.int32),
            pltpu.VMEM((bsz,), jnp.int32),
            pltpu.VMEM((bsz, d), jnp.float32),
            pltpu.VMEM((bsz, d), jnp.float32),
            pltpu.VMEM_SHARED((n_acc, d), jnp.float32),
            pltpu.SemaphoreType.DMA,
            pltpu.SemaphoreType.DMA,
        ],
    )(x, src, dst)


def _mm_body(x_ref, p_ref, w_ref, b_ref, z_ref, s_ref):
    i = pl.program_id(0)
    g = pl.num_programs(0)
    rst = x_ref[...] + p_ref[0] + p_ref[1]
    z = lax.dot_general(rst, w_ref[...], (((1,), (1,)), ((), ())),
                        preferred_element_type=jnp.float32)
    z = z + b_ref[...][None, :]
    z_ref[...] = z

    @pl.when(i == 0)
    def _():
        s_ref[...] = jnp.zeros_like(s_ref)

    colsum = jnp.sum(z, axis=0, keepdims=True)
    colsq = jnp.sum(z * z, axis=0, keepdims=True)
    s_ref[...] += jnp.concatenate([colsum, colsq], axis=0)


def _layer_mm(x, p, W, b):
    n, d = x.shape
    hdim = W.shape[0]
    r = 2000
    grid = (n // r,)
    return pl.pallas_call(
        _mm_body,
        grid=grid,
        in_specs=[
            pl.BlockSpec((r, d), lambda i: (i, 0)),
            pl.BlockSpec((NC, r, d), lambda i: (0, i, 0)),
            pl.BlockSpec((hdim, d), lambda i: (0, 0)),
            pl.BlockSpec((hdim,), lambda i: (0,)),
        ],
        out_specs=[
            pl.BlockSpec((r, hdim), lambda i: (i, 0)),
            pl.BlockSpec((2, hdim), lambda i: (0, 0)),
        ],
        out_shape=[
            jax.ShapeDtypeStruct((n, hdim), jnp.float32),
            jax.ShapeDtypeStruct((2, hdim), jnp.float32),
        ],
    )(x, p, W, b)


def _bn_body(z_ref, s_ref, g_ref, be_ref, o_ref, *, n):
    mu = s_ref[0:1, :] * (1.0 / n)
    ex2 = s_ref[1:2, :] * (1.0 / n)
    var = ex2 - mu * mu
    inv = lax.rsqrt(var + 1e-5)
    zn = (z_ref[...] - mu) * (inv * g_ref[...][None, :]) + be_ref[...][None, :]
    o_ref[...] = jnp.maximum(zn, 0.0)


def _bn_relu(z, stats, g, be):
    n, hdim = z.shape
    r = 2000
    return pl.pallas_call(
        functools.partial(_bn_body, n=n),
        grid=(n // r,),
        in_specs=[
            pl.BlockSpec((r, hdim), lambda i: (i, 0)),
            pl.BlockSpec((2, hdim), lambda i: (0, 0)),
            pl.BlockSpec((hdim,), lambda i: (0,)),
            pl.BlockSpec((hdim,), lambda i: (0,)),
        ],
        out_specs=pl.BlockSpec((r, hdim), lambda i: (i, 0)),
        out_shape=jax.ShapeDtypeStruct((n, hdim), jnp.float32),
    )(z, stats, g, be)


def _head_body(h_ref, w1_ref, b1_ref, w2_ref, b2_ref, o_ref, acc_ref, *, n):
    i = pl.program_id(0)
    g = pl.num_programs(0)

    @pl.when(i == 0)
    def _():
        acc_ref[...] = jnp.zeros_like(acc_ref)

    acc_ref[...] += jnp.sum(h_ref[...], axis=0, keepdims=True)

    @pl.when(i == g - 1)
    def _():
        hg = acc_ref[...] * (1.0 / n)
        y = lax.dot_general(hg, w1_ref[...], (((1,), (1,)), ((), ())),
                            preferred_element_type=jnp.float32)
        y = y + b1_ref[...][None, :]
        y = jnp.where(y > 0, y, jnp.exp(y) - 1.0)
        y2 = lax.dot_general(y, w2_ref[...], (((1,), (1,)), ((), ())),
                             preferred_element_type=jnp.float32)
        y2 = y2 + b2_ref[...][None, :]
        m = jnp.max(y2, axis=0, keepdims=True)
        lse = m + jnp.log(jnp.sum(jnp.exp(y2 - m), axis=0, keepdims=True))
        o_ref[...] = y2 - lse


def _head(hfin, fc1W, fc1b, fc2W, fc2b):
    n, hdim = hfin.shape
    cdim = fc2W.shape[0]
    r = 2000
    return pl.pallas_call(
        functools.partial(_head_body, n=n),
        grid=(n // r,),
        in_specs=[
            pl.BlockSpec((r, hdim), lambda i: (i, 0)),
            pl.BlockSpec((hdim, hdim), lambda i: (0, 0)),
            pl.BlockSpec((hdim,), lambda i: (0,)),
            pl.BlockSpec((cdim, hdim), lambda i: (0, 0)),
            pl.BlockSpec((cdim,), lambda i: (0,)),
        ],
        out_specs=pl.BlockSpec((1, cdim), lambda i: (0, 0)),
        out_shape=jax.ShapeDtypeStruct((1, cdim), jnp.float32),
        scratch_shapes=[pltpu.VMEM((1, hdim), jnp.float32)],
    )(hfin, fc1W, fc1b, fc2W, fc2b)


def kernel(h, edge_index, W0, b0, g0, be0, W1, b1, g1, be1, W2, b2, g2, be2,
           W3, b3, g3, be3, fc1W, fc1b, fc2W, fc2b):
    n = h.shape[0]
    e = edge_index.shape[1]
    bsz = 80
    nb = -(-e // (NW * bsz))
    epw = nb * bsz
    pad = NW * epw - e
    src = jnp.concatenate([edge_index[0], jnp.zeros((pad,), edge_index.dtype)])
    dst = jnp.concatenate([edge_index[1], jnp.full((pad,), n, edge_index.dtype)])
    x = h
    for W, b, g, be in ((W0, b0, g0, be0), (W1, b1, g1, be1),
                        (W2, b2, g2, be2), (W3, b3, g3, be3)):
        p = _aggregate(x, src, dst, nb, bsz)
        z, stats = _layer_mm(x, p, W, b)
        x = _bn_relu(z, stats, g, be)
    return _head(x, fc1W, fc1b, fc2W, fc2b)


# speedup vs baseline: 1.7410x; 1.0239x over previous
import functools

import jax
import jax.numpy as jnp
from jax import lax
from jax.experimental import pallas as pl
from jax.experimental.pallas import tpu as pltpu
from jax.experimental.pallas import tpu_sc as plsc

NC = 2
NS = 16
LANES = 16
NW = NC * NS


def _agg_body(x_hbm, src_hbm, dst_hbm, out_hbm,
              sidx, didx0, didx1, rows0, rows1, zbuf, acc,
              sem0, sem1, zsem, osem,
              *, bsz, nb, stride, wlen, zch):
    c = lax.axis_index("c")
    s = lax.axis_index("s")
    d = x_hbm.shape[1]
    rows = (rows0, rows1)
    didx = (didx0, didx1)
    sems = (sem0, sem1)
    wid = c * NS + s
    epw = nb * bsz
    ebase = wid * epw

    zero = jnp.zeros((LANES,), jnp.float32)
    for r in range(zch):
        for cc in range(d // LANES):
            zbuf[r, pl.ds(cc * LANES, LANES)] = zero
    base = s * stride
    nz = -(-wlen // zch)
    zcps = []
    for k in range(nz):
        off = min(k * zch, wlen - zch)
        cp = pltpu.make_async_copy(zbuf, acc.at[pl.ds(base + off, zch)], zsem)
        cp.start()
        zcps.append(cp)

    scp = pltpu.make_async_copy(src_hbm.at[pl.ds(ebase, epw)], sidx, osem)
    scp.start()
    scp.wait()

    def fire(j, b):
        pltpu.make_async_copy(
            x_hbm.at[sidx.at[pl.ds(j * bsz, bsz)]], rows[b], sems[b]).start()
        pltpu.make_async_copy(
            dst_hbm.at[pl.ds(ebase + j * bsz, bsz)], didx[b], sems[b]).start()

    def drain_scatter(j, b):
        pltpu.make_async_copy(
            x_hbm.at[sidx.at[pl.ds(j * bsz, bsz)]], rows[b], sems[b]).wait()
        pltpu.make_async_copy(
            dst_hbm.at[pl.ds(ebase + j * bsz, bsz)], didx[b], sems[b]).wait()
        pltpu.sync_copy(rows[b], acc.at[didx[b]], add=True)

    for b in range(2):
        fire(b, b)
    for cp in zcps:
        cp.wait()
    plsc.subcore_barrier()
    main = (nb - 2) // 2 * 2

    @pl.loop(0, main, step=2)
    def _(g):
        for b in range(2):
            j = g + b
            drain_scatter(j, b)
            fire(j + 2, b)

    for j in range(main, nb):
        drain_scatter(j, j % 2)
        if j + 2 < nb:
            fire(j + 2, j % 2)
    plsc.subcore_barrier()

    ocps = []
    for k in range(4):
        ch = wlen // 4
        cp = pltpu.make_async_copy(acc.at[pl.ds(base + k * ch, ch)],
                                   out_hbm.at[c, pl.ds(base + k * ch, ch)],
                                   osem)
        cp.start()
        ocps.append(cp)
    for cp in ocps:
        cp.wait()


def _aggregate(x, src, dst, nb, bsz):
    n, d = x.shape
    stride = (n // NS) // 8 * 8
    wlen = n - (NS - 1) * stride
    n_acc = n + 8
    zch = 40
    body = functools.partial(_agg_body, bsz=bsz, nb=nb,
                             stride=stride, wlen=wlen, zch=zch)
    return pl.kernel(
        body,
        out_type=jax.ShapeDtypeStruct((NC, n, d), jnp.float32),
        mesh=plsc.VectorSubcoreMesh(core_axis_name="c", subcore_axis_name="s"),
        scratch_types=[
            pltpu.VMEM((nb * bsz,), jnp.int32),
            pltpu.VMEM((bsz,), jnp.int32),
            pltpu.VMEM((bsz,), jnp.int32),
            pltpu.VMEM((bsz, d), jnp.float32),
            pltpu.VMEM((bsz, d), jnp.float32),
            pltpu.VMEM((zch, d), jnp.float32),
            pltpu.VMEM_SHARED((n_acc, d), jnp.float32),
            pltpu.SemaphoreType.DMA,
            pltpu.SemaphoreType.DMA,
            pltpu.SemaphoreType.DMA,
            pltpu.SemaphoreType.DMA,
        ],
    )(x, src, dst)


def _mm_body(x_ref, p_ref, w_ref, b_ref, z_ref, s_ref):
    i = pl.program_id(0)
    g = pl.num_programs(0)
    rst = x_ref[...] + p_ref[0] + p_ref[1]
    z = lax.dot_general(rst, w_ref[...], (((1,), (1,)), ((), ())),
                        preferred_element_type=jnp.float32)
    z = z + b_ref[...][None, :]
    z_ref[...] = z

    @pl.when(i == 0)
    def _():
        s_ref[...] = jnp.zeros_like(s_ref)

    colsum = jnp.sum(z, axis=0, keepdims=True)
    colsq = jnp.sum(z * z, axis=0, keepdims=True)
    s_ref[...] += jnp.concatenate([colsum, colsq], axis=0)


def _layer_mm(x, p, W, b):
    n, d = x.shape
    hdim = W.shape[0]
    r = 2000
    grid = (n // r,)
    return pl.pallas_call(
        _mm_body,
        grid=grid,
        in_specs=[
            pl.BlockSpec((r, d), lambda i: (i, 0)),
            pl.BlockSpec((NC, r, d), lambda i: (0, i, 0)),
            pl.BlockSpec((hdim, d), lambda i: (0, 0)),
            pl.BlockSpec((hdim,), lambda i: (0,)),
        ],
        out_specs=[
            pl.BlockSpec((r, hdim), lambda i: (i, 0)),
            pl.BlockSpec((2, hdim), lambda i: (0, 0)),
        ],
        out_shape=[
            jax.ShapeDtypeStruct((n, hdim), jnp.float32),
            jax.ShapeDtypeStruct((2, hdim), jnp.float32),
        ],
    )(x, p, W, b)


def _bn_body(z_ref, s_ref, g_ref, be_ref, o_ref, *, n):
    mu = s_ref[0:1, :] * (1.0 / n)
    ex2 = s_ref[1:2, :] * (1.0 / n)
    var = ex2 - mu * mu
    inv = lax.rsqrt(var + 1e-5)
    zn = (z_ref[...] - mu) * (inv * g_ref[...][None, :]) + be_ref[...][None, :]
    o_ref[...] = jnp.maximum(zn, 0.0)


def _bn_relu(z, stats, g, be):
    n, hdim = z.shape
    r = 2000
    return pl.pallas_call(
        functools.partial(_bn_body, n=n),
        grid=(n // r,),
        in_specs=[
            pl.BlockSpec((r, hdim), lambda i: (i, 0)),
            pl.BlockSpec((2, hdim), lambda i: (0, 0)),
            pl.BlockSpec((hdim,), lambda i: (0,)),
            pl.BlockSpec((hdim,), lambda i: (0,)),
        ],
        out_specs=pl.BlockSpec((r, hdim), lambda i: (i, 0)),
        out_shape=jax.ShapeDtypeStruct((n, hdim), jnp.float32),
    )(z, stats, g, be)


def _head_body(h_ref, w1_ref, b1_ref, w2_ref, b2_ref, o_ref, acc_ref, *, n):
    i = pl.program_id(0)
    g = pl.num_programs(0)

    @pl.when(i == 0)
    def _():
        acc_ref[...] = jnp.zeros_like(acc_ref)

    acc_ref[...] += jnp.sum(h_ref[...], axis=0, keepdims=True)

    @pl.when(i == g - 1)
    def _():
        hg = acc_ref[...] * (1.0 / n)
        y = lax.dot_general(hg, w1_ref[...], (((1,), (1,)), ((), ())),
                            preferred_element_type=jnp.float32)
        y = y + b1_ref[...][None, :]
        y = jnp.where(y > 0, y, jnp.exp(y) - 1.0)
        y2 = lax.dot_general(y, w2_ref[...], (((1,), (1,)), ((), ())),
                             preferred_element_type=jnp.float32)
        y2 = y2 + b2_ref[...][None, :]
        m = jnp.max(y2, axis=0, keepdims=True)
        lse = m + jnp.log(jnp.sum(jnp.exp(y2 - m), axis=0, keepdims=True))
        o_ref[...] = y2 - lse


def _head(hfin, fc1W, fc1b, fc2W, fc2b):
    n, hdim = hfin.shape
    cdim = fc2W.shape[0]
    r = 2000
    return pl.pallas_call(
        functools.partial(_head_body, n=n),
        grid=(n // r,),
        in_specs=[
            pl.BlockSpec((r, hdim), lambda i: (i, 0)),
            pl.BlockSpec((hdim, hdim), lambda i: (0, 0)),
            pl.BlockSpec((hdim,), lambda i: (0,)),
            pl.BlockSpec((cdim, hdim), lambda i: (0, 0)),
            pl.BlockSpec((cdim,), lambda i: (0,)),
        ],
        out_specs=pl.BlockSpec((1, cdim), lambda i: (0, 0)),
        out_shape=jax.ShapeDtypeStruct((1, cdim), jnp.float32),
        scratch_shapes=[pltpu.VMEM((1, hdim), jnp.float32)],
    )(hfin, fc1W, fc1b, fc2W, fc2b)


def kernel(h, edge_index, W0, b0, g0, be0, W1, b1, g1, be1, W2, b2, g2, be2,
           W3, b3, g3, be3, fc1W, fc1b, fc2W, fc2b):
    n = h.shape[0]
    e = edge_index.shape[1]
    bsz = 80
    nb = -(-e // (NW * bsz))
    epw = nb * bsz
    pad = NW * epw - e
    src = jnp.concatenate([edge_index[0], jnp.zeros((pad,), edge_index.dtype)])
    dst = jnp.concatenate([edge_index[1], jnp.full((pad,), n, edge_index.dtype)])
    x = h
    for W, b, g, be in ((W0, b0, g0, be0), (W1, b1, g1, be1),
                        (W2, b2, g2, be2), (W3, b3, g3, be3)):
        p = _aggregate(x, src, dst, nb, bsz)
        z, stats = _layer_mm(x, p, W, b)
        x = _bn_relu(z, stats, g, be)
    return _head(x, fc1W, fc1b, fc2W, fc2b)


# speedup vs baseline: 1.8001x; 1.0340x over previous
import functools

import jax
import jax.numpy as jnp
from jax import lax
from jax.experimental import pallas as pl
from jax.experimental.pallas import tpu as pltpu
from jax.experimental.pallas import tpu_sc as plsc

NC = 2
NS = 16
LANES = 16
NW = NC * NS


def _agg_body(x_hbm, src_hbm, dst_hbm, out_hbm,
              sidx, didx0, didx1, rows0, rows1, zbuf, acc,
              sem0, sem1, zsem, osem,
              *, bsz, nb, stride, wlen, zch):
    c = lax.axis_index("c")
    s = lax.axis_index("s")
    d = x_hbm.shape[1]
    rows = (rows0, rows1)
    didx = (didx0, didx1)
    sems = (sem0, sem1)
    wid = c * NS + s
    epw = nb * bsz
    ebase = wid * epw

    zero = jnp.zeros((LANES,), jnp.float32)
    for r in range(zch):
        for cc in range(d // LANES):
            zbuf[r, pl.ds(cc * LANES, LANES)] = zero
    base = s * stride
    nz = -(-wlen // zch)
    zcps = []
    for k in range(nz):
        off = min(k * zch, wlen - zch)
        cp = pltpu.make_async_copy(zbuf, acc.at[pl.ds(base + off, zch)], zsem)
        cp.start()
        zcps.append(cp)

    scp = pltpu.make_async_copy(src_hbm.at[pl.ds(ebase, epw)], sidx, osem)
    scp.start()
    scp.wait()

    def fire(j, b):
        pltpu.make_async_copy(
            x_hbm.at[sidx.at[pl.ds(j * bsz, bsz)]], rows[b], sems[b]).start()
        pltpu.make_async_copy(
            dst_hbm.at[pl.ds(ebase + j * bsz, bsz)], didx[b], sems[b]).start()

    def drain_scatter(j, b):
        pltpu.make_async_copy(
            x_hbm.at[sidx.at[pl.ds(j * bsz, bsz)]], rows[b], sems[b]).wait()
        pltpu.make_async_copy(
            dst_hbm.at[pl.ds(ebase + j * bsz, bsz)], didx[b], sems[b]).wait()
        pltpu.sync_copy(rows[b], acc.at[didx[b]], add=True)

    for b in range(2):
        fire(b, b)
    for cp in zcps:
        cp.wait()
    plsc.subcore_barrier()
    main = (nb - 2) // 2 * 2

    @pl.loop(0, main, step=2)
    def _(g):
        for b in range(2):
            j = g + b
            drain_scatter(j, b)
            fire(j + 2, b)

    for j in range(main, nb):
        drain_scatter(j, j % 2)
        if j + 2 < nb:
            fire(j + 2, j % 2)
    plsc.subcore_barrier()

    ocps = []
    for k in range(4):
        ch = wlen // 4
        cp = pltpu.make_async_copy(acc.at[pl.ds(base + k * ch, ch)],
                                   out_hbm.at[c, pl.ds(base + k * ch, ch)],
                                   osem)
        cp.start()
        ocps.append(cp)
    for cp in ocps:
        cp.wait()


def _aggregate(x, src, dst, nb, bsz):
    n, d = x.shape
    stride = (n // NS) // 8 * 8
    wlen = n - (NS - 1) * stride
    n_acc = n + 8
    zch = 40
    body = functools.partial(_agg_body, bsz=bsz, nb=nb,
                             stride=stride, wlen=wlen, zch=zch)
    return pl.kernel(
        body,
        out_type=jax.ShapeDtypeStruct((NC, n, d), jnp.float32),
        mesh=plsc.VectorSubcoreMesh(core_axis_name="c", subcore_axis_name="s"),
        scratch_types=[
            pltpu.VMEM((nb * bsz,), jnp.int32),
            pltpu.VMEM((bsz,), jnp.int32),
            pltpu.VMEM((bsz,), jnp.int32),
            pltpu.VMEM((bsz, d), jnp.float32),
            pltpu.VMEM((bsz, d), jnp.float32),
            pltpu.VMEM((zch, d), jnp.float32),
            pltpu.VMEM_SHARED((n_acc, d), jnp.float32),
            pltpu.SemaphoreType.DMA,
            pltpu.SemaphoreType.DMA,
            pltpu.SemaphoreType.DMA,
            pltpu.SemaphoreType.DMA,
        ],
    )(x, src, dst)


def _layer_body(x_ref, p_ref, w_ref, b_ref, g_ref, be_ref, o_ref,
                zs_ref, st_ref, *, n, r):
    ph = pl.program_id(0)
    i = pl.program_id(1)

    @pl.when(ph == 0)
    def _():
        @pl.when(i == 0)
        def _():
            st_ref[...] = jnp.zeros_like(st_ref)

        rst = x_ref[...] + p_ref[0] + p_ref[1]
        z = lax.dot_general(rst, w_ref[...], (((1,), (1,)), ((), ())),
                            preferred_element_type=jnp.float32)
        z = z + b_ref[...][None, :]
        zs_ref[pl.ds(i * r, r), :] = z
        colsum = jnp.sum(z, axis=0, keepdims=True)
        colsq = jnp.sum(z * z, axis=0, keepdims=True)
        st_ref[...] += jnp.concatenate([colsum, colsq], axis=0)

    @pl.when(ph == 1)
    def _():
        mu = st_ref[0:1, :] * (1.0 / n)
        ex2 = st_ref[1:2, :] * (1.0 / n)
        var = ex2 - mu * mu
        inv = lax.rsqrt(var + 1e-5)
        z = zs_ref[pl.ds(i * r, r), :]
        zn = (z - mu) * (inv * g_ref[...][None, :]) + be_ref[...][None, :]
        o_ref[...] = jnp.maximum(zn, 0.0)


def _layer_tc(x, p, W, b, g, be):
    n, d = x.shape
    hdim = W.shape[0]
    r = 2000
    return pl.pallas_call(
        functools.partial(_layer_body, n=n, r=r),
        grid=(2, n // r),
        in_specs=[
            pl.BlockSpec((r, d), lambda ph, i: (i * (1 - ph), 0)),
            pl.BlockSpec((NC, r, d), lambda ph, i: (0, i * (1 - ph), 0)),
            pl.BlockSpec((hdim, d), lambda ph, i: (0, 0)),
            pl.BlockSpec((hdim,), lambda ph, i: (0,)),
            pl.BlockSpec((hdim,), lambda ph, i: (0,)),
            pl.BlockSpec((hdim,), lambda ph, i: (0,)),
        ],
        out_specs=pl.BlockSpec((r, hdim), lambda ph, i: (i * ph, 0)),
        out_shape=jax.ShapeDtypeStruct((n, hdim), jnp.float32),
        scratch_shapes=[
            pltpu.VMEM((n, hdim), jnp.float32),
            pltpu.VMEM((2, hdim), jnp.float32),
        ],
    )(x, p, W, b, g, be)


def _head_body(h_ref, w1_ref, b1_ref, w2_ref, b2_ref, o_ref, acc_ref, *, n):
    i = pl.program_id(0)
    g = pl.num_programs(0)

    @pl.when(i == 0)
    def _():
        acc_ref[...] = jnp.zeros_like(acc_ref)

    acc_ref[...] += jnp.sum(h_ref[...], axis=0, keepdims=True)

    @pl.when(i == g - 1)
    def _():
        hg = acc_ref[...] * (1.0 / n)
        y = lax.dot_general(hg, w1_ref[...], (((1,), (1,)), ((), ())),
                            preferred_element_type=jnp.float32)
        y = y + b1_ref[...][None, :]
        y = jnp.where(y > 0, y, jnp.exp(y) - 1.0)
        y2 = lax.dot_general(y, w2_ref[...], (((1,), (1,)), ((), ())),
                             preferred_element_type=jnp.float32)
        y2 = y2 + b2_ref[...][None, :]
        m = jnp.max(y2, axis=0, keepdims=True)
        lse = m + jnp.log(jnp.sum(jnp.exp(y2 - m), axis=0, keepdims=True))
        o_ref[...] = y2 - lse


def _head(hfin, fc1W, fc1b, fc2W, fc2b):
    n, hdim = hfin.shape
    cdim = fc2W.shape[0]
    r = 2000
    return pl.pallas_call(
        functools.partial(_head_body, n=n),
        grid=(n // r,),
        in_specs=[
            pl.BlockSpec((r, hdim), lambda i: (i, 0)),
            pl.BlockSpec((hdim, hdim), lambda i: (0, 0)),
            pl.BlockSpec((hdim,), lambda i: (0,)),
            pl.BlockSpec((cdim, hdim), lambda i: (0, 0)),
            pl.BlockSpec((cdim,), lambda i: (0,)),
        ],
        out_specs=pl.BlockSpec((1, cdim), lambda i: (0, 0)),
        out_shape=jax.ShapeDtypeStruct((1, cdim), jnp.float32),
        scratch_shapes=[pltpu.VMEM((1, hdim), jnp.float32)],
    )(hfin, fc1W, fc1b, fc2W, fc2b)


def kernel(h, edge_index, W0, b0, g0, be0, W1, b1, g1, be1, W2, b2, g2, be2,
           W3, b3, g3, be3, fc1W, fc1b, fc2W, fc2b):
    n = h.shape[0]
    e = edge_index.shape[1]
    bsz = 80
    nb = -(-e // (NW * bsz))
    epw = nb * bsz
    pad = NW * epw - e
    src = jnp.concatenate([edge_index[0], jnp.zeros((pad,), edge_index.dtype)])
    dst = jnp.concatenate([edge_index[1], jnp.full((pad,), n, edge_index.dtype)])
    x = h
    for W, b, g, be in ((W0, b0, g0, be0), (W1, b1, g1, be1),
                        (W2, b2, g2, be2), (W3, b3, g3, be3)):
        p = _aggregate(x, src, dst, nb, bsz)
        x = _layer_tc(x, p, W, b, g, be)
    return _head(x, fc1W, fc1b, fc2W, fc2b)
